# initial kernel scaffold (unmeasured)
import jax
import jax.numpy as jnp
from jax import lax
from jax.experimental import pallas as pl
from jax.experimental.pallas import tpu as pltpu

N_DEV = 32


def kernel(x, w_mat):
    m_per, k = x.shape
    _, n = w_mat.shape
    n_per = n // N_DEV

    def body(x_ref, w_hbm, out_ref, wbuf, ybuf, wsems, send_sems, recv_sem):
        my = lax.axis_index("i")

        def start_w(s):
            c = lax.rem(my + s, N_DEV)
            cp = pltpu.make_async_copy(
                w_hbm.at[:, pl.ds(c * n_per, n_per)],
                wbuf.at[s % 2],
                wsems.at[s % 2],
            )
            cp.start()
            return cp

        w_cps = {0: start_w(0)}
        send_rdmas = [None, None]
        for s in range(N_DEV):
            w_cps[s].wait()
            if s + 1 < N_DEV:
                w_cps[s + 1] = start_w(s + 1)
            yblk = jnp.maximum(
                jnp.dot(x_ref[...], wbuf[s % 2],
                        preferred_element_type=jnp.float32),
                0.0,
            )
            if s == 0:
                out_ref[pl.ds(my * m_per, m_per), :] = yblk
            else:
                c = lax.rem(my + s, N_DEV)
                slot = s % 2
                if send_rdmas[slot] is not None:
                    send_rdmas[slot].wait_send()
                ybuf[slot] = yblk
                rdma = pltpu.make_async_remote_copy(
                    src_ref=ybuf.at[slot],
                    dst_ref=out_ref.at[pl.ds(my * m_per, m_per), :],
                    send_sem=send_sems.at[slot],
                    recv_sem=recv_sem,
                    device_id=(c,),
                    device_id_type=pl.DeviceIdType.MESH,
                )
                rdma.start()
                send_rdmas[slot] = rdma

        for r in send_rdmas:
            if r is not None:
                r.wait_send()

        recv_wait = pltpu.make_async_remote_copy(
            src_ref=ybuf.at[0],
            dst_ref=out_ref.at[pl.ds(0, m_per), :],
            send_sem=send_sems.at[0],
            recv_sem=recv_sem,
            device_id=(my,),
            device_id_type=pl.DeviceIdType.MESH,
        )
        for _ in range(N_DEV - 1):
            recv_wait.wait_recv()

    return pl.pallas_call(
        body,
        out_shape=jax.ShapeDtypeStruct((N_DEV * m_per, n_per), jnp.float32),
        in_specs=[
            pl.BlockSpec(memory_space=pltpu.VMEM),
            pl.BlockSpec(memory_space=pltpu.ANY),
        ],
        out_specs=pl.BlockSpec(memory_space=pltpu.VMEM),
        scratch_shapes=[
            pltpu.VMEM((2, k, n_per), x.dtype),
            pltpu.VMEM((2, m_per, n_per), jnp.float32),
            pltpu.SemaphoreType.DMA((2,)),
            pltpu.SemaphoreType.DMA((2,)),
            pltpu.SemaphoreType.DMA,
        ],
    )(x, w_mat)


# baseline (device time: 97216 ns/iter reference)
import jax
import jax.numpy as jnp
from jax import lax
from jax.experimental import pallas as pl
from jax.experimental.pallas import tpu as pltpu

N_DEV = 32


def kernel(x, w_mat):
    m_per, k = x.shape
    _, n = w_mat.shape
    n_per = n // N_DEV

    def body(x_ref, w_hbm, out_ref, wbuf, ybuf, wsems, send_sems, recv_sem):
        my = lax.axis_index("i")

        barrier_sem = pltpu.get_barrier_semaphore()
        for p in range(N_DEV):
            if p == 0:
                continue
            peer = lax.rem(my + p, N_DEV)
            pl.semaphore_signal(
                barrier_sem, inc=1,
                device_id=(peer,), device_id_type=pl.DeviceIdType.MESH,
            )
        pl.semaphore_wait(barrier_sem, N_DEV - 1)

        def start_w(s):
            c = lax.rem(my + s, N_DEV)
            cp = pltpu.make_async_copy(
                w_hbm.at[:, pl.ds(c * n_per, n_per)],
                wbuf.at[s % 2],
                wsems.at[s % 2],
            )
            cp.start()
            return cp

        w_cps = {0: start_w(0)}
        send_rdmas = [None, None]
        for s in range(N_DEV):
            w_cps[s].wait()
            if s + 1 < N_DEV:
                w_cps[s + 1] = start_w(s + 1)
            yblk = jnp.maximum(
                jnp.dot(x_ref[...], wbuf[s % 2],
                        preferred_element_type=jnp.float32),
                0.0,
            )
            if s == 0:
                out_ref[pl.ds(my * m_per, m_per), :] = yblk
            else:
                c = lax.rem(my + s, N_DEV)
                slot = s % 2
                if send_rdmas[slot] is not None:
                    send_rdmas[slot].wait_send()
                ybuf[slot] = yblk
                rdma = pltpu.make_async_remote_copy(
                    src_ref=ybuf.at[slot],
                    dst_ref=out_ref.at[pl.ds(my * m_per, m_per), :],
                    send_sem=send_sems.at[slot],
                    recv_sem=recv_sem,
                    device_id=(c,),
                    device_id_type=pl.DeviceIdType.MESH,
                )
                rdma.start()
                send_rdmas[slot] = rdma

        for r in send_rdmas:
            if r is not None:
                r.wait_send()

        recv_wait = pltpu.make_async_remote_copy(
            src_ref=ybuf.at[0],
            dst_ref=out_ref.at[pl.ds(0, m_per), :],
            send_sem=send_sems.at[0],
            recv_sem=recv_sem,
            device_id=(my,),
            device_id_type=pl.DeviceIdType.MESH,
        )
        for _ in range(N_DEV - 1):
            recv_wait.wait_recv()

    return pl.pallas_call(
        body,
        out_shape=jax.ShapeDtypeStruct((N_DEV * m_per, n_per), jnp.float32),
        in_specs=[
            pl.BlockSpec(memory_space=pltpu.VMEM),
            pl.BlockSpec(memory_space=pl.ANY),
        ],
        out_specs=pl.BlockSpec(memory_space=pltpu.VMEM),
        scratch_shapes=[
            pltpu.VMEM((2, k, n_per), x.dtype),
            pltpu.VMEM((2, m_per, n_per), jnp.float32),
            pltpu.SemaphoreType.DMA((2,)),
            pltpu.SemaphoreType.DMA((2,)),
            pltpu.SemaphoreType.DMA,
        ],
        compiler_params=pltpu.CompilerParams(collective_id=0),
    )(x, w_mat)


# device time: 90718 ns/iter; 1.0716x vs baseline; 1.0716x over previous
import jax
import jax.numpy as jnp
from jax import lax
from jax.experimental import pallas as pl
from jax.experimental.pallas import tpu as pltpu

N_DEV = 32


def kernel(x, w_mat):
    m_per, k = x.shape
    _, n = w_mat.shape
    n_per = n // N_DEV

    def body(x_ref, w_hbm, out_ref, xbf, wbuf, ybuf, recvbuf,
             wsems, send_sems, recv_sems):
        my = lax.axis_index("i")

        barrier_sem = pltpu.get_barrier_semaphore()
        for p in range(1, N_DEV):
            peer = lax.rem(my + p, N_DEV)
            pl.semaphore_signal(
                barrier_sem, inc=1,
                device_id=(peer,), device_id_type=pl.DeviceIdType.MESH,
            )
        pl.semaphore_wait(barrier_sem, N_DEV - 1)

        xbf[...] = x_ref[...].astype(jnp.bfloat16)

        def start_w(s):
            c = lax.rem(my + s, N_DEV)
            cp = pltpu.make_async_copy(
                w_hbm.at[:, pl.ds(c * n_per, n_per)],
                wbuf.at[s % 2],
                wsems.at[s % 2],
            )
            cp.start()
            return cp

        w_cps = {0: start_w(0)}
        send_rdmas = [None, None]
        for s in range(N_DEV):
            w_cps[s].wait()
            if s + 1 < N_DEV:
                w_cps[s + 1] = start_w(s + 1)
            yblk = jnp.maximum(
                jnp.dot(xbf[...], wbuf[s % 2].astype(jnp.bfloat16),
                        preferred_element_type=jnp.float32),
                0.0,
            )
            if s == 0:
                out_ref[pl.ds(my * m_per, m_per), :] = yblk
            else:
                c = lax.rem(my + s, N_DEV)
                slot = s % 2
                if send_rdmas[slot] is not None:
                    send_rdmas[slot].wait_send()
                ybuf[slot] = yblk.astype(jnp.bfloat16)
                rdma = pltpu.make_async_remote_copy(
                    src_ref=ybuf.at[slot],
                    dst_ref=recvbuf.at[my],
                    send_sem=send_sems.at[slot],
                    recv_sem=recv_sems.at[my],
                    device_id=(c,),
                    device_id_type=pl.DeviceIdType.MESH,
                )
                rdma.start()
                send_rdmas[slot] = rdma

        for s in range(1, N_DEV):
            src = lax.rem(my - s + N_DEV, N_DEV)
            recv_wait = pltpu.make_async_remote_copy(
                src_ref=ybuf.at[0],
                dst_ref=recvbuf.at[src],
                send_sem=send_sems.at[0],
                recv_sem=recv_sems.at[src],
                device_id=(my,),
                device_id_type=pl.DeviceIdType.MESH,
            )
            recv_wait.wait_recv()
            out_ref[pl.ds(src * m_per, m_per), :] = (
                recvbuf[src].astype(jnp.float32)
            )

        for r in send_rdmas:
            if r is not None:
                r.wait_send()

    return pl.pallas_call(
        body,
        out_shape=jax.ShapeDtypeStruct((N_DEV * m_per, n_per), jnp.float32),
        in_specs=[
            pl.BlockSpec(memory_space=pltpu.VMEM),
            pl.BlockSpec(memory_space=pl.ANY),
        ],
        out_specs=pl.BlockSpec(memory_space=pltpu.VMEM),
        scratch_shapes=[
            pltpu.VMEM((m_per, k), jnp.bfloat16),
            pltpu.VMEM((2, k, n_per), w_mat.dtype),
            pltpu.VMEM((2, m_per, n_per), jnp.bfloat16),
            pltpu.VMEM((N_DEV, m_per, n_per), jnp.bfloat16),
            pltpu.SemaphoreType.DMA((2,)),
            pltpu.SemaphoreType.DMA((2,)),
            pltpu.SemaphoreType.DMA((N_DEV,)),
        ],
        compiler_params=pltpu.CompilerParams(collective_id=0),
    )(x, w_mat)


# device time: 76978 ns/iter; 1.2629x vs baseline; 1.1785x over previous
import os

import jax
import jax.numpy as jnp
from jax import lax
from jax.experimental import pallas as pl
from jax.experimental.pallas import tpu as pltpu

N_DEV = 32
SUPER = 4
N_SUP = N_DEV // SUPER
KSPLIT = 2
SLOTS = 8

_SKIP_SEND = os.environ.get("KERNEL_SKIP_SEND") == "1"
_SKIP_WDMA = os.environ.get("KERNEL_SKIP_WDMA") == "1"
_SKIP_DOT = os.environ.get("KERNEL_SKIP_DOT") == "1"


def kernel(x, w_mat):
    m_per, k = x.shape
    _, n = w_mat.shape
    n_per = n // N_DEV
    n_sup = n_per * SUPER
    kh = k // KSPLIT
    n_subs = N_SUP * KSPLIT

    def body(x_ref, w_hbm, out_ref, xbf, wbuf, ysup, recvbuf,
             wsems, send_sems, recv_sems):
        my = lax.axis_index("i")
        my_g = lax.div(my, SUPER)

        barrier_sem = pltpu.get_barrier_semaphore()
        for p in range(1, N_DEV):
            peer = lax.rem(my + p, N_DEV)
            pl.semaphore_signal(
                barrier_sem, inc=1,
                device_id=(peer,), device_id_type=pl.DeviceIdType.MESH,
            )
        pl.semaphore_wait(barrier_sem, N_DEV - 1)

        xbf[...] = x_ref[...].astype(jnp.bfloat16)

        def start_wsub(u):
            t, kk = divmod(u, KSPLIT)
            g = lax.rem(my_g + t, N_SUP)
            cp = pltpu.make_async_copy(
                w_hbm.at[pl.ds(kk * kh, kh), pl.ds(g * n_sup, n_sup)],
                wbuf.at[u % 2],
                wsems.at[u % 2],
            )
            cp.start()
            return cp

        w_cps = {}
        if not _SKIP_WDMA:
            w_cps[0] = start_wsub(0)
            w_cps[1] = start_wsub(1)

        send_rdmas = [None] * SLOTS
        n_sent = 0
        for t in range(N_SUP):
            g = lax.rem(my_g + t, N_SUP)
            acc = None
            for kk in range(KSPLIT):
                u = t * KSPLIT + kk
                if not _SKIP_WDMA:
                    w_cps[u].wait()
                if _SKIP_DOT:
                    part = jnp.zeros((m_per, n_sup), jnp.float32)
                else:
                    part = jnp.dot(
                        xbf[:, kk * kh:(kk + 1) * kh],
                        wbuf[u % 2].astype(jnp.bfloat16),
                        preferred_element_type=jnp.float32,
                    )
                if not _SKIP_WDMA and u + 2 < n_subs:
                    w_cps[u + 2] = start_wsub(u + 2)
                acc = part if acc is None else acc + part
            yblk = jnp.maximum(acc, 0.0)

            ybuf_slot = t % 2
            if not _SKIP_SEND:
                for j in range(SUPER):
                    sl = (t * SUPER + j) % SLOTS
                    if send_rdmas[sl] is not None:
                        send_rdmas[sl].wait_send()
                        send_rdmas[sl] = None
            ysup[ybuf_slot] = yblk.astype(jnp.bfloat16)

            if t == 0:
                off_own = lax.rem(my, SUPER) * n_per
                out_ref[pl.ds(my * m_per, m_per), :] = (
                    ysup[ybuf_slot, :, pl.ds(off_own, n_per)]
                    .astype(jnp.float32)
                )

            if _SKIP_SEND:
                continue

            for j in range(SUPER):
                if t == 0 and j == 0:
                    continue
                rj = lax.rem(my + j, SUPER)
                r = g * SUPER + rj
                sl = (t * SUPER + j) % SLOTS
                rdma = pltpu.make_async_remote_copy(
                    src_ref=ysup.at[ybuf_slot, :, pl.ds(rj * n_per, n_per)],
                    dst_ref=recvbuf.at[my],
                    send_sem=send_sems.at[sl],
                    recv_sem=recv_sems.at[my],
                    device_id=(r,),
                    device_id_type=pl.DeviceIdType.MESH,
                )
                rdma.start()
                send_rdmas[sl] = rdma
                n_sent += 1

        if _SKIP_SEND:
            return

        for t in range(N_SUP):
            q = lax.rem(my_g - t + N_SUP, N_SUP)
            for j in range(SUPER):
                if t == 0 and j == 0:
                    continue
                src = q * SUPER + lax.rem(my - j + SUPER, SUPER)
                recv_wait = pltpu.make_async_remote_copy(
                    src_ref=recvbuf.at[0],
                    dst_ref=recvbuf.at[src],
                    send_sem=send_sems.at[0],
                    recv_sem=recv_sems.at[src],
                    device_id=(my,),
                    device_id_type=pl.DeviceIdType.MESH,
                )
                recv_wait.wait_recv()
                out_ref[pl.ds(src * m_per, m_per), :] = (
                    recvbuf[src].astype(jnp.float32)
                )

        for r in send_rdmas:
            if r is not None:
                r.wait_send()

    return pl.pallas_call(
        body,
        out_shape=jax.ShapeDtypeStruct((N_DEV * m_per, n_per), jnp.float32),
        in_specs=[
            pl.BlockSpec(memory_space=pltpu.VMEM),
            pl.BlockSpec(memory_space=pl.ANY),
        ],
        out_specs=pl.BlockSpec(memory_space=pltpu.VMEM),
        scratch_shapes=[
            pltpu.VMEM((m_per, k), jnp.bfloat16),
            pltpu.VMEM((2, kh, n_sup), w_mat.dtype),
            pltpu.VMEM((2, m_per, n_sup), jnp.bfloat16),
            pltpu.VMEM((N_DEV, m_per, n_per), jnp.bfloat16),
            pltpu.SemaphoreType.DMA((2,)),
            pltpu.SemaphoreType.DMA((SLOTS,)),
            pltpu.SemaphoreType.DMA((N_DEV,)),
        ],
        compiler_params=pltpu.CompilerParams(
            collective_id=0,
            vmem_limit_bytes=100 * 1024 * 1024,
        ),
    )(x, w_mat)


# device time: 73656 ns/iter; 1.3199x vs baseline; 1.0451x over previous
import os

import jax
import jax.numpy as jnp
from jax import lax
from jax.experimental import pallas as pl
from jax.experimental.pallas import tpu as pltpu

N_DEV = 32
SUPER = 8
N_SUP = N_DEV // SUPER
KSPLIT = 4
SLOTS = 16

_SKIP_SEND = os.environ.get("KERNEL_SKIP_SEND") == "1"
_SKIP_WDMA = os.environ.get("KERNEL_SKIP_WDMA") == "1"
_SKIP_DOT = os.environ.get("KERNEL_SKIP_DOT") == "1"


def kernel(x, w_mat):
    m_per, k = x.shape
    _, n = w_mat.shape
    n_per = n // N_DEV
    n_sup = n_per * SUPER
    kh = k // KSPLIT
    n_subs = N_SUP * KSPLIT

    def body(x_ref, w_hbm, out_ref, xbf, wbuf, ysup, recvbuf,
             wsems, send_sems, recv_sems):
        my = lax.axis_index("i")
        my_g = lax.div(my, SUPER)

        barrier_sem = pltpu.get_barrier_semaphore()
        for p in range(1, N_DEV):
            peer = lax.rem(my + p, N_DEV)
            pl.semaphore_signal(
                barrier_sem, inc=1,
                device_id=(peer,), device_id_type=pl.DeviceIdType.MESH,
            )
        pl.semaphore_wait(barrier_sem, N_DEV - 1)

        xbf[...] = x_ref[...].astype(jnp.bfloat16)

        def start_wsub(u):
            t, kk = divmod(u, KSPLIT)
            g = lax.rem(my_g + t, N_SUP)
            khh = kh // 2
            cps = []
            for h in range(2):
                cp = pltpu.make_async_copy(
                    w_hbm.at[pl.ds(kk * kh + h * khh, khh),
                             pl.ds(g * n_sup, n_sup)],
                    wbuf.at[u % 2, pl.ds(h * khh, khh)],
                    wsems.at[u % 2, h],
                )
                cp.start()
                cps.append(cp)
            return cps

        w_cps = {}
        if not _SKIP_WDMA:
            w_cps[0] = start_wsub(0)
            w_cps[1] = start_wsub(1)

        send_rdmas = [None] * SLOTS
        n_sent = 0
        for t in range(N_SUP):
            g = lax.rem(my_g + t, N_SUP)
            acc = None
            for kk in range(KSPLIT):
                u = t * KSPLIT + kk
                if not _SKIP_WDMA:
                    for cp in w_cps[u]:
                        cp.wait()
                if _SKIP_DOT:
                    part = jnp.zeros((m_per, n_sup), jnp.float32)
                else:
                    part = jnp.dot(
                        xbf[:, kk * kh:(kk + 1) * kh],
                        wbuf[u % 2].astype(jnp.bfloat16),
                        preferred_element_type=jnp.float32,
                    )
                if not _SKIP_WDMA and u + 2 < n_subs:
                    w_cps[u + 2] = start_wsub(u + 2)
                acc = part if acc is None else acc + part
            yblk = jnp.maximum(acc, 0.0)

            ybuf_slot = t % 2
            if not _SKIP_SEND:
                for j in range(SUPER):
                    sl = (t * SUPER + j) % SLOTS
                    if send_rdmas[sl] is not None:
                        send_rdmas[sl].wait_send()
                        send_rdmas[sl] = None
            ysup[ybuf_slot] = yblk.astype(jnp.bfloat16)

            if t == 0:
                off_own = lax.rem(my, SUPER) * n_per
                recvbuf[my, :, :] = ysup[ybuf_slot, :, pl.ds(off_own, n_per)]

            if _SKIP_SEND:
                continue

            for j in range(SUPER):
                if t == 0 and j == 0:
                    continue
                rj = lax.rem(my + j, SUPER)
                r = g * SUPER + rj
                sl = (t * SUPER + j) % SLOTS
                rdma = pltpu.make_async_remote_copy(
                    src_ref=ysup.at[ybuf_slot, :, pl.ds(rj * n_per, n_per)],
                    dst_ref=recvbuf.at[my],
                    send_sem=send_sems.at[sl],
                    recv_sem=recv_sems.at[my],
                    device_id=(r,),
                    device_id_type=pl.DeviceIdType.MESH,
                )
                rdma.start()
                send_rdmas[sl] = rdma
                n_sent += 1

        if not _SKIP_SEND:
            for off in range(1, N_DEV):
                src = lax.rem(my + off, N_DEV)
                recv_wait = pltpu.make_async_remote_copy(
                    src_ref=recvbuf.at[0],
                    dst_ref=recvbuf.at[src],
                    send_sem=send_sems.at[0],
                    recv_sem=recv_sems.at[src],
                    device_id=(my,),
                    device_id_type=pl.DeviceIdType.MESH,
                )
                recv_wait.wait_recv()

        out_ref[...] = jnp.reshape(
            recvbuf[...], (N_DEV * m_per, n_per)
        ).astype(jnp.float32)

        if not _SKIP_SEND:
            for r in send_rdmas:
                if r is not None:
                    r.wait_send()

    return pl.pallas_call(
        body,
        out_shape=jax.ShapeDtypeStruct((N_DEV * m_per, n_per), jnp.float32),
        in_specs=[
            pl.BlockSpec(memory_space=pltpu.VMEM),
            pl.BlockSpec(memory_space=pl.ANY),
        ],
        out_specs=pl.BlockSpec(memory_space=pltpu.VMEM),
        scratch_shapes=[
            pltpu.VMEM((m_per, k), jnp.bfloat16),
            pltpu.VMEM((2, kh, n_sup), w_mat.dtype),
            pltpu.VMEM((2, m_per, n_sup), jnp.bfloat16),
            pltpu.VMEM((N_DEV, m_per, n_per), jnp.bfloat16),
            pltpu.SemaphoreType.DMA((2, 2)),
            pltpu.SemaphoreType.DMA((SLOTS,)),
            pltpu.SemaphoreType.DMA((N_DEV,)),
        ],
        compiler_params=pltpu.CompilerParams(
            collective_id=0,
            vmem_limit_bytes=100 * 1024 * 1024,
        ),
    )(x, w_mat)
